# baseline (speedup 1.0000x reference)
import jax
import jax.numpy as jnp
from jax.experimental import pallas as pl
from jax.experimental.pallas import tpu as pltpu

_NB = 8
_INV_H = 2.5
_E = 8
_TM = 512


def _silu(v):
    return v * jax.nn.sigmoid(v)


def _spline_slab(u):
    u2 = u * u
    u3 = u2 * u
    p0 = u3 * (1.0 / 6.0)
    p1 = (-3.0 * u3 + 12.0 * u2 - 12.0 * u + 4.0) * (1.0 / 6.0)
    p2 = (3.0 * u3 - 24.0 * u2 + 60.0 * u - 44.0) * (1.0 / 6.0)
    v4 = 4.0 - u
    p3 = v4 * v4 * v4 * (1.0 / 6.0)
    m = jnp.where(u < 1.0, p0, jnp.where(u < 2.0, p1, jnp.where(u < 3.0, p2, p3)))
    return jnp.where((u >= 0.0) & (u < 4.0), m, jnp.zeros_like(u))


def _spline_rows_bf16(xt):
    s0 = (xt + 1.0) * _INV_H + 3.0
    slabs = [_spline_slab(s0 - float(n)).astype(jnp.bfloat16) for n in range(_NB)]
    st = jnp.stack(slabs, axis=1)
    return st.reshape(xt.shape[0] * _NB, xt.shape[1])


def _fuse_body(w1_ref, s1_ref, w2_ref, s2_ref, w3_ref, s3_ref,
               o1_ref, o2_ref, o3_ref):
    o1_ref[...] = (w1_ref[...] * s1_ref[...]).astype(jnp.bfloat16)
    o2_ref[...] = (w2_ref[...] * s2_ref[...]).astype(jnp.bfloat16)
    o3_ref[...] = (w3_ref[...] * s3_ref[...]).astype(jnp.bfloat16)


def _prep_body(xt_ref, gw_ref, sx_ref, bx_ref, ww_ref):
    xt = xt_ref[...]
    sx_ref[...] = _silu(xt).astype(jnp.bfloat16)
    bx_ref[...] = _spline_rows_bf16(xt)
    logits = jnp.dot(gw_ref[...].astype(jnp.bfloat16), xt.astype(jnp.bfloat16),
                     preferred_element_type=jnp.float32)
    ne = logits.shape[0]
    row = jax.lax.broadcasted_iota(jnp.int32, logits.shape, 0)
    m1 = jnp.max(logits, axis=0, keepdims=True)
    i1 = jnp.min(jnp.where(logits == m1, row, ne), axis=0, keepdims=True)
    l2 = jnp.where(row == i1, jnp.float32(-jnp.inf), logits)
    m2 = jnp.max(l2, axis=0, keepdims=True)
    i2 = jnp.min(jnp.where(l2 == m2, row, ne), axis=0, keepdims=True)
    e2 = jnp.exp(m2 - m1)
    denom = 1.0 + e2
    ww_ref[...] = (jnp.where(row == i1, 1.0 / denom, 0.0)
                   + jnp.where(row == i2, e2 / denom, 0.0))


def _k12_body(sx_ref, bx_ref, w1s_ref, w1b_ref, w2s_ref, w2b_ref, hp_ref):
    f32 = jnp.float32
    bf = jnp.bfloat16
    sx = sx_ref[...]
    bx = bx_ref[...]
    h1 = (jnp.dot(w1b_ref[0].astype(bf), sx, preferred_element_type=f32)
          + jnp.dot(w1s_ref[0], bx, preferred_element_type=f32))
    h2 = (jnp.dot(w2b_ref[0].astype(bf), sx, preferred_element_type=f32)
          + jnp.dot(w2s_ref[0], bx, preferred_element_type=f32))
    hp_ref[0] = h1 * h2


def _k3_body(hp_ref, ww_ref, w3s_ref, w3b_ref, y_ref):
    e = pl.program_id(0)
    f32 = jnp.float32
    hp = hp_ref[0]
    sh = _silu(hp).astype(jnp.bfloat16)
    b2 = _spline_rows_bf16(hp)
    y = (jnp.dot(w3b_ref[0].astype(jnp.bfloat16), sh, preferred_element_type=f32)
         + jnp.dot(w3s_ref[0], b2, preferred_element_type=f32))
    ww = ww_ref[...]
    row = jax.lax.broadcasted_iota(jnp.int32, ww.shape, 0)
    wrow = jnp.sum(jnp.where(row == e, ww, 0.0), axis=0, keepdims=True)
    y_ref[0] = y * wrow


def _comb_body(y_ref, o_ref):
    acc = y_ref[0]
    for e in range(1, _E):
        acc = acc + y_ref[e]
    o_ref[...] = acc


def kernel(x, gate_w, w1_base, w1_spline, w1_scaler, w2_base, w2_spline,
           w2_scaler, w3_base, w3_spline, w3_scaler, grid_in, grid_ff):
    B, S, D = x.shape
    E, F, _ = w1_base.shape
    NB = _NB
    TM = min(_TM, S)
    R = S // TM

    xt = x.reshape(S, D).T
    w1n = w1_spline.reshape(E, F, D * NB)
    w2n = w2_spline.reshape(E, F, D * NB)
    w3n = w3_spline.reshape(E, D, F * NB)
    s1r = jnp.repeat(w1_scaler, NB, axis=2)
    s2r = jnp.repeat(w2_scaler, NB, axis=2)
    s3r = jnp.repeat(w3_scaler, NB, axis=2)

    OC = 4

    def specf(o, k):
        return pl.BlockSpec((1, o // OC, k), lambda e, c: (e, c, 0))

    W1s, W2s, W3s = pl.pallas_call(
        _fuse_body,
        grid=(E, OC),
        in_specs=[specf(F, D * NB), specf(F, D * NB),
                  specf(F, D * NB), specf(F, D * NB),
                  specf(D, F * NB), specf(D, F * NB)],
        out_specs=[specf(F, D * NB), specf(F, D * NB), specf(D, F * NB)],
        out_shape=[jax.ShapeDtypeStruct((E, F, D * NB), jnp.bfloat16),
                   jax.ShapeDtypeStruct((E, F, D * NB), jnp.bfloat16),
                   jax.ShapeDtypeStruct((E, D, F * NB), jnp.bfloat16)],
        compiler_params=pltpu.CompilerParams(
            dimension_semantics=("parallel", "arbitrary")),
    )(w1n, s1r, w2n, s2r, w3n, s3r)

    SX, BX, WW = pl.pallas_call(
        _prep_body,
        grid=(R,),
        in_specs=[pl.BlockSpec((D, TM), lambda r: (0, r)),
                  pl.BlockSpec((E, D), lambda r: (0, 0))],
        out_specs=[pl.BlockSpec((D, TM), lambda r: (0, r)),
                   pl.BlockSpec((NB * D, TM), lambda r: (0, r)),
                   pl.BlockSpec((E, TM), lambda r: (0, r))],
        out_shape=[jax.ShapeDtypeStruct((D, S), jnp.bfloat16),
                   jax.ShapeDtypeStruct((NB * D, S), jnp.bfloat16),
                   jax.ShapeDtypeStruct((E, S), jnp.float32)],
        compiler_params=pltpu.CompilerParams(
            dimension_semantics=("arbitrary",)),
    )(xt, gate_w)

    HP = pl.pallas_call(
        _k12_body,
        grid=(E, R),
        in_specs=[pl.BlockSpec((D, TM), lambda e, r: (0, r)),
                  pl.BlockSpec((NB * D, TM), lambda e, r: (0, r)),
                  pl.BlockSpec((1, F, NB * D), lambda e, r: (e, 0, 0)),
                  pl.BlockSpec((1, F, D), lambda e, r: (e, 0, 0)),
                  pl.BlockSpec((1, F, NB * D), lambda e, r: (e, 0, 0)),
                  pl.BlockSpec((1, F, D), lambda e, r: (e, 0, 0))],
        out_specs=pl.BlockSpec((1, F, TM), lambda e, r: (e, 0, r)),
        out_shape=jax.ShapeDtypeStruct((E, F, S), jnp.float32),
        compiler_params=pltpu.CompilerParams(
            dimension_semantics=("parallel", "arbitrary")),
    )(SX, BX, W1s, w1_base, W2s, w2_base)

    YP = pl.pallas_call(
        _k3_body,
        grid=(E, R),
        in_specs=[pl.BlockSpec((1, F, TM), lambda e, r: (e, 0, r)),
                  pl.BlockSpec((E, TM), lambda e, r: (0, r)),
                  pl.BlockSpec((1, D, NB * F), lambda e, r: (e, 0, 0)),
                  pl.BlockSpec((1, D, F), lambda e, r: (e, 0, 0))],
        out_specs=pl.BlockSpec((1, D, TM), lambda e, r: (e, 0, r)),
        out_shape=jax.ShapeDtypeStruct((E, D, S), jnp.float32),
        compiler_params=pltpu.CompilerParams(
            dimension_semantics=("parallel", "arbitrary")),
    )(HP, WW, W3s, w3_base)

    outt = pl.pallas_call(
        _comb_body,
        grid=(R,),
        in_specs=[pl.BlockSpec((E, D, TM), lambda r: (0, 0, r))],
        out_specs=pl.BlockSpec((D, TM), lambda r: (0, r)),
        out_shape=jax.ShapeDtypeStruct((D, S), jnp.float32),
        compiler_params=pltpu.CompilerParams(
            dimension_semantics=("arbitrary",)),
    )(YP)

    return outt.T.reshape(B, S, D)


# speedup vs baseline: 2.3699x; 2.3699x over previous
import functools

import jax
import jax.numpy as jnp
from jax.experimental import pallas as pl
from jax.experimental.pallas import tpu as pltpu
from jax.experimental.pallas import tpu_sc as plsc

_NB = 8
_INV_H = 2.5
_E = 8
_TM = 512
_T = 128
_NTP = 48


def _silu(v):
    return v * jax.nn.sigmoid(v)


def _spline_slab(u):
    u2 = u * u
    u3 = u2 * u
    p0 = u3 * (1.0 / 6.0)
    p1 = (-3.0 * u3 + 12.0 * u2 - 12.0 * u + 4.0) * (1.0 / 6.0)
    p2 = (3.0 * u3 - 24.0 * u2 + 60.0 * u - 44.0) * (1.0 / 6.0)
    v4 = 4.0 - u
    p3 = v4 * v4 * v4 * (1.0 / 6.0)
    m = jnp.where(u < 1.0, p0, jnp.where(u < 2.0, p1, jnp.where(u < 3.0, p2, p3)))
    return jnp.where((u >= 0.0) & (u < 4.0), m, jnp.zeros_like(u))


def _spline_blocks_bf16(x):
    s0 = (x + 1.0) * _INV_H + 3.0
    return [_spline_slab(s0 - float(n)).astype(jnp.bfloat16) for n in range(_NB)]


def _fuse_body(w1_ref, s1_ref, w2_ref, s2_ref, w3_ref, s3_ref,
               o1_ref, o2_ref, o3_ref):
    o1_ref[...] = (w1_ref[...] * s1_ref[...]).astype(jnp.bfloat16)
    o2_ref[...] = (w2_ref[...] * s2_ref[...]).astype(jnp.bfloat16)
    o3_ref[...] = (w3_ref[...] * s3_ref[...]).astype(jnp.bfloat16)


def _prep_body(x_ref, gwt_ref, i12_ref, p12_ref):
    xv = x_ref[...]
    logits = jnp.dot(xv.astype(jnp.bfloat16), gwt_ref[...].astype(jnp.bfloat16),
                     preferred_element_type=jnp.float32)
    ne = logits.shape[1]
    lane = jax.lax.broadcasted_iota(jnp.int32, logits.shape, 1)
    m1 = jnp.max(logits, axis=1, keepdims=True)
    i1 = jnp.min(jnp.where(logits == m1, lane, ne), axis=1, keepdims=True)
    l2 = jnp.where(lane == i1, jnp.float32(-jnp.inf), logits)
    m2 = jnp.max(l2, axis=1, keepdims=True)
    i2 = jnp.min(jnp.where(l2 == m2, lane, ne), axis=1, keepdims=True)
    e2 = jnp.exp(m2 - m1)
    denom = 1.0 + e2
    i12_ref[...] = jnp.concatenate([i1, i2], axis=1)
    p12_ref[...] = jnp.concatenate([1.0 / denom, e2 / denom], axis=1)


def _meta_body(i12_ref, dest_ref, gidx_ref, te_ref):
    f32 = jnp.float32
    hi = jax.lax.Precision.HIGHEST
    i12 = i12_ref[...]
    s = i12.shape[0]
    np_ = gidx_ref.shape[1]
    lane8 = jax.lax.broadcasted_iota(jnp.int32, (s, 8), 1)
    oh0 = (i12[:, 0:1] == lane8).astype(f32)
    oh1 = (i12[:, 1:2] == lane8).astype(f32)
    oh = jnp.concatenate([oh0, oh1], axis=1)
    ch = 128
    r_i = jax.lax.broadcasted_iota(jnp.int32, (ch, ch), 0)
    c_i = jax.lax.broadcasted_iota(jnp.int32, (ch, ch), 1)
    tril = (c_i < r_i).astype(f32)
    carry = jnp.zeros((1, 16), f32)
    prefs = []
    for c in range(s // ch):
        blk = oh[c * ch:(c + 1) * ch, :]
        prefs.append(jnp.dot(tril, blk, precision=hi,
                             preferred_element_type=f32) + carry)
        carry = carry + jnp.sum(blk, axis=0, keepdims=True)
    pref = jnp.concatenate(prefs, axis=0)
    c0 = carry[:, 0:8]
    counts = c0 + carry[:, 8:16]
    ceilc = jnp.floor((counts + (_T - 1)) * (1.0 / _T)) * float(_T)
    e_r = jax.lax.broadcasted_iota(jnp.int32, (8, 8), 0)
    e_c = jax.lax.broadcasted_iota(jnp.int32, (8, 8), 1)
    u8 = (e_r < e_c).astype(f32)
    pad_base = jnp.dot(jnp.broadcast_to(ceilc, (8, 8)), u8, precision=hi,
                       preferred_element_type=f32)[0:1, :]
    pad_next = pad_base + ceilc
    d0 = jnp.sum(oh0 * (pad_base + pref[:, 0:8]), axis=1, keepdims=True)
    d1 = jnp.sum(oh1 * (pad_base + c0 + pref[:, 8:16]), axis=1, keepdims=True)
    destc = jnp.concatenate([d0, d1], axis=0)
    dest_ref[...] = jnp.broadcast_to(destc, (2 * s, 8)).astype(jnp.int32)
    tok = (jax.lax.broadcasted_iota(jnp.int32, (8, 2 * s), 1) % s).astype(f32)
    gparts = []
    dch = 256
    for c in range(np_ // dch):
        dcol = (jax.lax.broadcasted_iota(jnp.int32, (1, dch), 1)
                + c * dch).astype(f32)
        mask = (destc == dcol).astype(f32)
        gparts.append(jnp.dot(tok, mask, precision=hi,
                              preferred_element_type=f32)[0:1, :])
    gidx_ref[...] = jnp.concatenate(gparts, axis=1).astype(jnp.int32)
    tile_j = jax.lax.broadcasted_iota(jnp.int32, (_NTP, 8), 0).astype(f32)
    cmp = (jnp.broadcast_to(pad_next, (_NTP, 8)) <= tile_j * float(_T))
    te = jnp.clip(jnp.sum(cmp.astype(jnp.int32), axis=1, keepdims=True), 0, 7)
    te_ref[...] = jnp.broadcast_to(te, (_NTP, 8))


def _kan_body(te_ref, xg_ref, w1s_ref, w1b_ref, w2s_ref, w2b_ref,
              w3s_ref, w3b_ref, yg_ref):
    f32 = jnp.float32
    xv = xg_ref[...]
    sx = _silu(xv).astype(jnp.bfloat16)
    bx = jnp.concatenate(_spline_blocks_bf16(xv), axis=1)
    h1 = (jnp.dot(sx, w1b_ref[0], preferred_element_type=f32)
          + jnp.dot(bx, w1s_ref[0], preferred_element_type=f32))
    h2 = (jnp.dot(sx, w2b_ref[0], preferred_element_type=f32)
          + jnp.dot(bx, w2s_ref[0], preferred_element_type=f32))
    hp = h1 * h2
    sh = _silu(hp).astype(jnp.bfloat16)
    b2 = jnp.concatenate(_spline_blocks_bf16(hp), axis=1)
    yg_ref[...] = (jnp.dot(sh, w3b_ref[0], preferred_element_type=f32)
                   + jnp.dot(b2, w3s_ref[0], preferred_element_type=f32))


def _comb_body(g0_ref, g1_ref, p_ref, o_ref):
    p = p_ref[...]
    lane = jax.lax.broadcasted_iota(jnp.int32, p.shape, 1)
    p0 = jnp.sum(jnp.where(lane == 0, p, 0.0), axis=1, keepdims=True)
    p1 = jnp.sum(jnp.where(lane == 1, p, 0.0), axis=1, keepdims=True)
    o_ref[...] = p0 * g0_ref[0] + p1 * g1_ref[0]


def _sc_gather(table, idx):
    n = idx.shape[0]
    d = table.shape[1]
    info = plsc.get_sparse_core_info()
    nc = info.num_cores
    nw = nc * info.num_subcores
    bpw = n // nw
    mesh = plsc.VectorSubcoreMesh(core_axis_name="c", subcore_axis_name="s")

    @functools.partial(
        pl.kernel, mesh=mesh,
        out_type=jax.ShapeDtypeStruct((n, d), table.dtype),
        scratch_types=[pltpu.VMEM((bpw,), jnp.int32),
                       pltpu.VMEM((bpw, d), table.dtype),
                       pltpu.SemaphoreType.DMA],
    )
    def k(table_hbm, idx_hbm, out_hbm, idx_v, rows_v, sem):
        wid = jax.lax.axis_index("s") * nc + jax.lax.axis_index("c")
        base = wid * bpw
        pltpu.sync_copy(idx_hbm.at[pl.ds(base, bpw)], idx_v)
        pltpu.async_copy(table_hbm.at[idx_v], rows_v, sem).wait()
        pltpu.sync_copy(rows_v, out_hbm.at[pl.ds(base, bpw)])

    return k(table, idx)


def kernel(x, gate_w, w1_base, w1_spline, w1_scaler, w2_base, w2_spline,
           w2_scaler, w3_base, w3_spline, w3_scaler, grid_in, grid_ff):
    B, S, D = x.shape
    E, F, _ = w1_base.shape
    NB = _NB
    TM = min(_TM, S)
    R = S // TM
    NP = 2 * S + E * _T
    NT = NP // _T
    xf = x.reshape(S, D)

    w1t = jnp.transpose(w1_spline, (0, 3, 2, 1))
    w2t = jnp.transpose(w2_spline, (0, 3, 2, 1))
    w3t = jnp.transpose(w3_spline, (0, 3, 2, 1))
    s1t = jnp.transpose(w1_scaler, (0, 2, 1))
    s2t = jnp.transpose(w2_scaler, (0, 2, 1))
    s3t = jnp.transpose(w3_scaler, (0, 2, 1))
    b1t = jnp.transpose(w1_base, (0, 2, 1)).astype(jnp.bfloat16)
    b2t = jnp.transpose(w2_base, (0, 2, 1)).astype(jnp.bfloat16)
    b3t = jnp.transpose(w3_base, (0, 2, 1)).astype(jnp.bfloat16)

    def spec4(i, o):
        return pl.BlockSpec((1, 1, i, o), lambda e, n: (e, n, 0, 0))

    def spec3(i, o):
        return pl.BlockSpec((1, i, o), lambda e, n: (e, 0, 0))

    W1s, W2s, W3s = pl.pallas_call(
        _fuse_body,
        grid=(E, NB),
        in_specs=[spec4(D, F), spec3(D, F), spec4(D, F), spec3(D, F),
                  spec4(F, D), spec3(F, D)],
        out_specs=[spec4(D, F), spec4(D, F), spec4(F, D)],
        out_shape=[jax.ShapeDtypeStruct((E, NB, D, F), jnp.bfloat16),
                   jax.ShapeDtypeStruct((E, NB, D, F), jnp.bfloat16),
                   jax.ShapeDtypeStruct((E, NB, F, D), jnp.bfloat16)],
        compiler_params=pltpu.CompilerParams(
            dimension_semantics=("parallel", "arbitrary")),
    )(w1t, s1t, w2t, s2t, w3t, s3t)
    W1s = W1s.reshape(E, NB * D, F)
    W2s = W2s.reshape(E, NB * D, F)
    W3s = W3s.reshape(E, NB * F, D)

    I12, P12 = pl.pallas_call(
        _prep_body,
        grid=(R,),
        in_specs=[pl.BlockSpec((TM, D), lambda r: (r, 0)),
                  pl.BlockSpec((D, E), lambda r: (0, 0))],
        out_specs=[pl.BlockSpec((TM, 2), lambda r: (r, 0)),
                   pl.BlockSpec((TM, 2), lambda r: (r, 0))],
        out_shape=[jax.ShapeDtypeStruct((S, 2), jnp.int32),
                   jax.ShapeDtypeStruct((S, 2), jnp.float32)],
        compiler_params=pltpu.CompilerParams(
            dimension_semantics=("arbitrary",)),
    )(xf, gate_w.T)

    DEST, GIDX, TE = pl.pallas_call(
        _meta_body,
        grid=(1,),
        in_specs=[pl.BlockSpec((S, 2), lambda i: (0, 0))],
        out_specs=[pl.BlockSpec((2 * S, 8), lambda i: (0, 0)),
                   pl.BlockSpec((1, NP), lambda i: (0, 0)),
                   pl.BlockSpec((_NTP, 8), lambda i: (0, 0))],
        out_shape=[jax.ShapeDtypeStruct((2 * S, 8), jnp.int32),
                   jax.ShapeDtypeStruct((1, NP), jnp.int32),
                   jax.ShapeDtypeStruct((_NTP, 8), jnp.int32)],
        compiler_params=pltpu.CompilerParams(
            dimension_semantics=("arbitrary",)),
    )(I12)
    destflat = DEST[:, 0]
    gidx = GIDX.reshape(NP)

    XG = _sc_gather(xf, gidx)

    grid_spec = pltpu.PrefetchScalarGridSpec(
        num_scalar_prefetch=1,
        grid=(NT,),
        in_specs=[pl.BlockSpec((_T, D), lambda j, te: (j, 0)),
                  pl.BlockSpec((1, NB * D, F), lambda j, te: (te[j, 0], 0, 0)),
                  pl.BlockSpec((1, D, F), lambda j, te: (te[j, 0], 0, 0)),
                  pl.BlockSpec((1, NB * D, F), lambda j, te: (te[j, 0], 0, 0)),
                  pl.BlockSpec((1, D, F), lambda j, te: (te[j, 0], 0, 0)),
                  pl.BlockSpec((1, NB * F, D), lambda j, te: (te[j, 0], 0, 0)),
                  pl.BlockSpec((1, F, D), lambda j, te: (te[j, 0], 0, 0))],
        out_specs=pl.BlockSpec((_T, D), lambda j, te: (j, 0)),
    )
    YG = pl.pallas_call(
        _kan_body,
        grid_spec=grid_spec,
        out_shape=jax.ShapeDtypeStruct((NP, D), jnp.float32),
        compiler_params=pltpu.CompilerParams(
            dimension_semantics=("arbitrary",)),
    )(TE, XG, W1s, b1t, W2s, b2t, W3s, b3t)

    G = _sc_gather(YG, destflat)
    G3 = G.reshape(2, S, D)

    out = pl.pallas_call(
        _comb_body,
        grid=(R,),
        in_specs=[pl.BlockSpec((1, TM, D), lambda r: (0, r, 0)),
                  pl.BlockSpec((1, TM, D), lambda r: (1, r, 0)),
                  pl.BlockSpec((TM, 2), lambda r: (r, 0))],
        out_specs=pl.BlockSpec((TM, D), lambda r: (r, 0)),
        out_shape=jax.ShapeDtypeStruct((S, D), jnp.float32),
        compiler_params=pltpu.CompilerParams(
            dimension_semantics=("arbitrary",)),
    )(G3, G3, P12)

    return out.reshape(B, S, D)


# speedup vs baseline: 2.4094x; 1.0167x over previous
import functools

import jax
import jax.numpy as jnp
from jax.experimental import pallas as pl
from jax.experimental.pallas import tpu as pltpu
from jax.experimental.pallas import tpu_sc as plsc

_NB = 8
_INV_H = 2.5
_E = 8
_TM = 512
_T = 128
_NTP = 48


def _silu(v):
    return v * jax.nn.sigmoid(v)


def _spline_slab(u):
    u2 = u * u
    u3 = u2 * u
    p0 = u3 * (1.0 / 6.0)
    p1 = (-3.0 * u3 + 12.0 * u2 - 12.0 * u + 4.0) * (1.0 / 6.0)
    p2 = (3.0 * u3 - 24.0 * u2 + 60.0 * u - 44.0) * (1.0 / 6.0)
    v4 = 4.0 - u
    p3 = v4 * v4 * v4 * (1.0 / 6.0)
    m = jnp.where(u < 1.0, p0, jnp.where(u < 2.0, p1, jnp.where(u < 3.0, p2, p3)))
    return jnp.where((u >= 0.0) & (u < 4.0), m, jnp.zeros_like(u))


def _spline_blocks_bf16(x):
    s0 = (x + 1.0) * _INV_H + 3.0
    return [_spline_slab(s0 - float(n)).astype(jnp.bfloat16) for n in range(_NB)]


def _fuse_body(w1_ref, s1_ref, w2_ref, s2_ref, w3_ref, s3_ref,
               o1_ref, o2_ref, o3_ref):
    f32 = jnp.float32
    o1_ref[...] = (w1_ref[...].astype(f32) * s1_ref[...]).astype(jnp.bfloat16)
    o2_ref[...] = (w2_ref[...].astype(f32) * s2_ref[...]).astype(jnp.bfloat16)
    o3_ref[...] = (w3_ref[...].astype(f32) * s3_ref[...]).astype(jnp.bfloat16)


def _prep_body(x_ref, gwt_ref, i12_ref, p12_ref):
    xv = x_ref[...]
    logits = jnp.dot(xv.astype(jnp.bfloat16), gwt_ref[...].astype(jnp.bfloat16),
                     preferred_element_type=jnp.float32)
    ne = logits.shape[1]
    lane = jax.lax.broadcasted_iota(jnp.int32, logits.shape, 1)
    m1 = jnp.max(logits, axis=1, keepdims=True)
    i1 = jnp.min(jnp.where(logits == m1, lane, ne), axis=1, keepdims=True)
    l2 = jnp.where(lane == i1, jnp.float32(-jnp.inf), logits)
    m2 = jnp.max(l2, axis=1, keepdims=True)
    i2 = jnp.min(jnp.where(l2 == m2, lane, ne), axis=1, keepdims=True)
    e2 = jnp.exp(m2 - m1)
    denom = 1.0 + e2
    i12_ref[...] = jnp.concatenate([i1, i2], axis=1)
    p12_ref[...] = jnp.concatenate([1.0 / denom, e2 / denom], axis=1)


def _meta_body(i12_ref, dest_ref, gidx_ref, te_ref):
    f32 = jnp.float32
    hi = jax.lax.Precision.HIGHEST
    i12 = i12_ref[...]
    s = i12.shape[0]
    np_ = gidx_ref.shape[1]
    lane8 = jax.lax.broadcasted_iota(jnp.int32, (s, 8), 1)
    oh0 = (i12[:, 0:1] == lane8).astype(f32)
    oh1 = (i12[:, 1:2] == lane8).astype(f32)
    oh = jnp.concatenate([oh0, oh1], axis=1)
    ch = 128
    r_i = jax.lax.broadcasted_iota(jnp.int32, (ch, ch), 0)
    c_i = jax.lax.broadcasted_iota(jnp.int32, (ch, ch), 1)
    tril = (c_i < r_i).astype(f32)
    carry = jnp.zeros((1, 16), f32)
    prefs = []
    for c in range(s // ch):
        blk = oh[c * ch:(c + 1) * ch, :]
        prefs.append(jnp.dot(tril, blk, precision=hi,
                             preferred_element_type=f32) + carry)
        carry = carry + jnp.sum(blk, axis=0, keepdims=True)
    pref = jnp.concatenate(prefs, axis=0)
    c0 = carry[:, 0:8]
    counts = c0 + carry[:, 8:16]
    ceilc = jnp.floor((counts + (_T - 1)) * (1.0 / _T)) * float(_T)
    e_r = jax.lax.broadcasted_iota(jnp.int32, (8, 8), 0)
    e_c = jax.lax.broadcasted_iota(jnp.int32, (8, 8), 1)
    u8 = (e_r < e_c).astype(f32)
    pad_base = jnp.dot(jnp.broadcast_to(ceilc, (8, 8)), u8, precision=hi,
                       preferred_element_type=f32)[0:1, :]
    pad_next = pad_base + ceilc
    d0 = jnp.sum(oh0 * (pad_base + pref[:, 0:8]), axis=1, keepdims=True)
    d1 = jnp.sum(oh1 * (pad_base + c0 + pref[:, 8:16]), axis=1, keepdims=True)
    destc = jnp.concatenate([d0, d1], axis=0)
    dest_ref[...] = jnp.broadcast_to(destc, (2 * s, 8)).astype(jnp.int32)
    tok = (jax.lax.broadcasted_iota(jnp.int32, (8, 2 * s), 1) % s).astype(f32)
    gparts = []
    dch = 256
    for c in range(np_ // dch):
        dcol = (jax.lax.broadcasted_iota(jnp.int32, (1, dch), 1)
                + c * dch).astype(f32)
        mask = (destc == dcol).astype(f32)
        gparts.append(jnp.dot(tok, mask, precision=hi,
                              preferred_element_type=f32)[0:1, :])
    gidx_ref[...] = jnp.concatenate(gparts, axis=1).astype(jnp.int32)
    tile_j = jax.lax.broadcasted_iota(jnp.int32, (_NTP, 8), 0).astype(f32)
    cmp = (jnp.broadcast_to(pad_next, (_NTP, 8)) <= tile_j * float(_T))
    te = jnp.clip(jnp.sum(cmp.astype(jnp.int32), axis=1, keepdims=True), 0, 7)
    te_ref[...] = jnp.broadcast_to(te, (_NTP, 8))


def _kan_body(te_ref, xg_ref, w1s_ref, w1b_ref, w2s_ref, w2b_ref,
              w3s_ref, w3b_ref, yg_ref):
    f32 = jnp.float32
    xv = xg_ref[...]
    sx = _silu(xv).astype(jnp.bfloat16)
    bx = jnp.concatenate(_spline_blocks_bf16(xv), axis=1)
    h1 = (jnp.dot(sx, w1b_ref[0], preferred_element_type=f32)
          + jnp.dot(bx, w1s_ref[0], preferred_element_type=f32))
    h2 = (jnp.dot(sx, w2b_ref[0], preferred_element_type=f32)
          + jnp.dot(bx, w2s_ref[0], preferred_element_type=f32))
    hp = h1 * h2
    sh = _silu(hp).astype(jnp.bfloat16)
    b2 = jnp.concatenate(_spline_blocks_bf16(hp), axis=1)
    yg_ref[...] = (jnp.dot(sh, w3b_ref[0], preferred_element_type=f32)
                   + jnp.dot(b2, w3s_ref[0], preferred_element_type=f32))


def _comb_body(g0_ref, g1_ref, p_ref, o_ref):
    p = p_ref[...]
    lane = jax.lax.broadcasted_iota(jnp.int32, p.shape, 1)
    p0 = jnp.sum(jnp.where(lane == 0, p, 0.0), axis=1, keepdims=True)
    p1 = jnp.sum(jnp.where(lane == 1, p, 0.0), axis=1, keepdims=True)
    o_ref[...] = p0 * g0_ref[0] + p1 * g1_ref[0]


def _sc_gather(table, idx):
    n = idx.shape[0]
    d = table.shape[1]
    info = plsc.get_sparse_core_info()
    nc = info.num_cores
    nw = nc * info.num_subcores
    bpw = n // nw
    mesh = plsc.VectorSubcoreMesh(core_axis_name="c", subcore_axis_name="s")

    @functools.partial(
        pl.kernel, mesh=mesh,
        out_type=jax.ShapeDtypeStruct((n, d), table.dtype),
        scratch_types=[pltpu.VMEM((bpw,), jnp.int32),
                       pltpu.VMEM((bpw, d), table.dtype),
                       pltpu.SemaphoreType.DMA],
    )
    def k(table_hbm, idx_hbm, out_hbm, idx_v, rows_v, sem):
        wid = jax.lax.axis_index("s") * nc + jax.lax.axis_index("c")
        base = wid * bpw
        pltpu.sync_copy(idx_hbm.at[pl.ds(base, bpw)], idx_v)
        pltpu.async_copy(table_hbm.at[idx_v], rows_v, sem).wait()
        pltpu.sync_copy(rows_v, out_hbm.at[pl.ds(base, bpw)])

    return k(table, idx)


def kernel(x, gate_w, w1_base, w1_spline, w1_scaler, w2_base, w2_spline,
           w2_scaler, w3_base, w3_spline, w3_scaler, grid_in, grid_ff):
    B, S, D = x.shape
    E, F, _ = w1_base.shape
    NB = _NB
    TM = min(_TM, S)
    R = S // TM
    NP = 2 * S + E * _T
    NT = NP // _T
    xf = x.reshape(S, D)

    w1t = jnp.transpose(w1_spline.astype(jnp.bfloat16), (0, 3, 2, 1))
    w2t = jnp.transpose(w2_spline.astype(jnp.bfloat16), (0, 3, 2, 1))
    w3t = jnp.transpose(w3_spline.astype(jnp.bfloat16), (0, 3, 2, 1))
    s1t = jnp.transpose(w1_scaler, (0, 2, 1))
    s2t = jnp.transpose(w2_scaler, (0, 2, 1))
    s3t = jnp.transpose(w3_scaler, (0, 2, 1))
    b1t = jnp.transpose(w1_base, (0, 2, 1)).astype(jnp.bfloat16)
    b2t = jnp.transpose(w2_base, (0, 2, 1)).astype(jnp.bfloat16)
    b3t = jnp.transpose(w3_base, (0, 2, 1)).astype(jnp.bfloat16)

    def spec4(i, o):
        return pl.BlockSpec((1, 1, i, o), lambda e, n: (e, n, 0, 0))

    def spec3(i, o):
        return pl.BlockSpec((1, i, o), lambda e, n: (e, 0, 0))

    W1s, W2s, W3s = pl.pallas_call(
        _fuse_body,
        grid=(E, NB),
        in_specs=[spec4(D, F), spec3(D, F), spec4(D, F), spec3(D, F),
                  spec4(F, D), spec3(F, D)],
        out_specs=[spec4(D, F), spec4(D, F), spec4(F, D)],
        out_shape=[jax.ShapeDtypeStruct((E, NB, D, F), jnp.bfloat16),
                   jax.ShapeDtypeStruct((E, NB, D, F), jnp.bfloat16),
                   jax.ShapeDtypeStruct((E, NB, F, D), jnp.bfloat16)],
        compiler_params=pltpu.CompilerParams(
            dimension_semantics=("parallel", "arbitrary")),
    )(w1t, s1t, w2t, s2t, w3t, s3t)
    W1s = W1s.reshape(E, NB * D, F)
    W2s = W2s.reshape(E, NB * D, F)
    W3s = W3s.reshape(E, NB * F, D)

    I12, P12 = pl.pallas_call(
        _prep_body,
        grid=(R,),
        in_specs=[pl.BlockSpec((TM, D), lambda r: (r, 0)),
                  pl.BlockSpec((D, E), lambda r: (0, 0))],
        out_specs=[pl.BlockSpec((TM, 2), lambda r: (r, 0)),
                   pl.BlockSpec((TM, 2), lambda r: (r, 0))],
        out_shape=[jax.ShapeDtypeStruct((S, 2), jnp.int32),
                   jax.ShapeDtypeStruct((S, 2), jnp.float32)],
        compiler_params=pltpu.CompilerParams(
            dimension_semantics=("arbitrary",)),
    )(xf, gate_w.T)

    DEST, GIDX, TE = pl.pallas_call(
        _meta_body,
        grid=(1,),
        in_specs=[pl.BlockSpec((S, 2), lambda i: (0, 0))],
        out_specs=[pl.BlockSpec((2 * S, 8), lambda i: (0, 0)),
                   pl.BlockSpec((1, NP), lambda i: (0, 0)),
                   pl.BlockSpec((_NTP, 8), lambda i: (0, 0))],
        out_shape=[jax.ShapeDtypeStruct((2 * S, 8), jnp.int32),
                   jax.ShapeDtypeStruct((1, NP), jnp.int32),
                   jax.ShapeDtypeStruct((_NTP, 8), jnp.int32)],
        compiler_params=pltpu.CompilerParams(
            dimension_semantics=("arbitrary",)),
    )(I12)
    destflat = DEST[:, 0]
    gidx = GIDX.reshape(NP)

    XG = _sc_gather(xf, gidx)

    grid_spec = pltpu.PrefetchScalarGridSpec(
        num_scalar_prefetch=1,
        grid=(NT,),
        in_specs=[pl.BlockSpec((_T, D), lambda j, te: (j, 0)),
                  pl.BlockSpec((1, NB * D, F), lambda j, te: (te[j, 0], 0, 0)),
                  pl.BlockSpec((1, D, F), lambda j, te: (te[j, 0], 0, 0)),
                  pl.BlockSpec((1, NB * D, F), lambda j, te: (te[j, 0], 0, 0)),
                  pl.BlockSpec((1, D, F), lambda j, te: (te[j, 0], 0, 0)),
                  pl.BlockSpec((1, NB * F, D), lambda j, te: (te[j, 0], 0, 0)),
                  pl.BlockSpec((1, F, D), lambda j, te: (te[j, 0], 0, 0))],
        out_specs=pl.BlockSpec((_T, D), lambda j, te: (j, 0)),
    )
    YG = pl.pallas_call(
        _kan_body,
        grid_spec=grid_spec,
        out_shape=jax.ShapeDtypeStruct((NP, D), jnp.float32),
        compiler_params=pltpu.CompilerParams(
            dimension_semantics=("arbitrary",)),
    )(TE, XG, W1s, b1t, W2s, b2t, W3s, b3t)

    G = _sc_gather(YG, destflat)
    G3 = G.reshape(2, S, D)

    out = pl.pallas_call(
        _comb_body,
        grid=(R,),
        in_specs=[pl.BlockSpec((1, TM, D), lambda r: (0, r, 0)),
                  pl.BlockSpec((1, TM, D), lambda r: (1, r, 0)),
                  pl.BlockSpec((TM, 2), lambda r: (r, 0))],
        out_specs=pl.BlockSpec((TM, D), lambda r: (r, 0)),
        out_shape=jax.ShapeDtypeStruct((S, D), jnp.float32),
        compiler_params=pltpu.CompilerParams(
            dimension_semantics=("arbitrary",)),
    )(G3, G3, P12)

    return out.reshape(B, S, D)
